# Initial kernel scaffold; baseline (speedup 1.0000x reference)
import jax
import jax.numpy as jnp
from jax.experimental import pallas as pl


def kernel(pred_cloud, gt_cloud):
    raise NotImplementedError("write your pallas kernel here")


# speedup vs baseline: 26.9537x; 26.9537x over previous
import functools
import numpy as np
import jax
import jax.numpy as jnp
from jax import lax
from jax.experimental import pallas as pl
from jax.experimental.pallas import tpu as pltpu
from jax.experimental.pallas import tpu_sc as plsc

S = 64
NV = S * S * S
NPTS = 65536
NB = 8
NQ = 4
QSZ = NV // NQ
QPLANES = S // NQ
CHUNK = 1024
NCH = NPTS // CHUNK
VPC = CHUNK // 16

_LO = np.float32(-32.0)
_UP = np.float32(np.float32(31.0) + np.float32(1.0) - np.float32(1e-5))


def _grid_body(pred_ref, gt_ref, g1_ref, g2_ref, grid_v, pts_a, pts_b,
               sem0, sem1):
  cid = lax.axis_index("c")
  sid = lax.axis_index("s")
  wid = sid * 2 + cid
  batch = wid // NQ
  q = wid % NQ
  qlo = q * QPLANES
  i3 = lax.iota(jnp.int32, 16) * 3
  zeros16 = jnp.zeros((16,), jnp.float32)
  sems = (sem0, sem1)
  bufs = (pts_a, pts_b)

  for src, dst in ((pred_ref, g1_ref), (gt_ref, g2_ref)):

    @pl.loop(0, QSZ // 16, unroll=8)
    def _zero(i):
      grid_v[pl.ds(i * 16, 16)] = zeros16

    def start(ch, b):
      pltpu.async_copy(
          src.at[batch, pl.ds(ch * (CHUNK * 3), CHUNK * 3)],
          bufs[b], sems[b])

    def wait(b):
      pltpu.make_async_copy(
          src.at[batch, pl.ds(0, CHUNK * 3)], bufs[b], sems[b]).wait()

    def process(b):
      buf = bufs[b]

      @pl.loop(0, VPC, unroll=2)
      def _vec(v):
        ix = i3 + v * 48
        axes = []
        for d in (0, 1, 2):
          p = plsc.load_gather(buf, [ix + d])
          s = jnp.minimum(jnp.maximum(p * 32.0, _LO), _UP)
          ti = s.astype(jnp.int32)
          tf = ti.astype(jnp.float32)
          blt = s < tf
          frac = (s - tf) + blt.astype(jnp.float32)
          lo = (ti - blt.astype(jnp.int32)) + 32
          w1 = jnp.where(lo < 63, frac, jnp.float32(0.0))
          g1 = jnp.minimum(lo + 1, 63)
          axes.append((lo, g1, jnp.float32(1.0) - frac, w1))
        (lx, gx1, wx0, wx1), (ly, gy1, wy0, wy1), (lz, gz1, wz0, wz1) = axes

        xa = (lx - qlo) * 4096
        xb = (gx1 - qlo) * 4096
        m0 = (lx >= qlo) & (lx < qlo + QPLANES)
        m1 = (gx1 >= qlo) & (gx1 < qlo + QPLANES)
        ya = ly * 64
        yb = gy1 * 64
        yz = ((ya + lz, wy0 * wz0), (ya + gz1, wy0 * wz1),
              (yb + lz, wy1 * wz0), (yb + gz1, wy1 * wz1))
        for xp, wx, m in ((xa, wx0, m0), (xb, wx1, m1)):
          for yzp, wyz in yz:
            plsc.addupdate_scatter(grid_v, [xp + yzp], wx * wyz, mask=m)

    start(0, 0)

    @pl.loop(0, NCH // 2 - 1)
    def _pair(p):
      ch = p * 2
      start(ch + 1, 1)
      wait(0)
      process(0)
      start(ch + 2, 0)
      wait(1)
      process(1)

    start(NCH - 1, 1)
    wait(0)
    process(0)
    wait(1)
    process(1)

    pltpu.sync_copy(grid_v, dst.at[batch, pl.ds(q * QSZ, QSZ)])


@jax.jit
def _gridding(pred, gt):
  mesh = plsc.VectorSubcoreMesh(
      core_axis_name="c", subcore_axis_name="s", num_cores=2, num_subcores=16)
  return pl.kernel(
      _grid_body,
      out_type=(jax.ShapeDtypeStruct((NB, NV), jnp.float32),
                jax.ShapeDtypeStruct((NB, NV), jnp.float32)),
      mesh=mesh,
      compiler_params=pltpu.CompilerParams(needs_layout_passes=False),
      scratch_types=[
          pltpu.VMEM((QSZ,), jnp.float32),
          pltpu.VMEM((CHUNK * 3,), jnp.float32),
          pltpu.VMEM((CHUNK * 3,), jnp.float32),
          pltpu.SemaphoreType.DMA,
          pltpu.SemaphoreType.DMA,
      ],
  )(pred, gt)


def kernel(pred_cloud, gt_cloud):
  pred = pred_cloud.reshape(NB, NPTS * 3)
  gt = gt_cloud.reshape(NB, NPTS * 3)
  return _gridding(pred, gt)


# speedup vs baseline: 26.9784x; 1.0009x over previous
import functools
import numpy as np
import jax
import jax.numpy as jnp
from jax import lax
from jax.experimental import pallas as pl
from jax.experimental.pallas import tpu as pltpu
from jax.experimental.pallas import tpu_sc as plsc

S = 64
NV = S * S * S
NPTS = 65536
NB = 8
NQ = 4
QSZ = NV // NQ
QPLANES = S // NQ
CHUNK = 1024
NCH = NPTS // CHUNK
VPC = CHUNK // 16

_LO = np.float32(-32.0)
_UP = np.float32(np.float32(31.0) + np.float32(1.0) - np.float32(1e-5))


def _grid_body(pred_ref, gt_ref, g1_ref, g2_ref, grid_v, pts_a, pts_b,
               sem0, sem1):
  cid = lax.axis_index("c")
  sid = lax.axis_index("s")
  wid = sid * 2 + cid
  batch = wid // NQ
  q = wid % NQ
  qlo = q * QPLANES
  i3 = lax.iota(jnp.int32, 16) * 3
  zeros16 = jnp.zeros((16,), jnp.float32)
  sems = (sem0, sem1)
  bufs = (pts_a, pts_b)

  for src, dst in ((pred_ref, g1_ref), (gt_ref, g2_ref)):

    @pl.loop(0, QSZ // 16, unroll=8)
    def _zero(i):
      grid_v[pl.ds(i * 16, 16)] = zeros16

    def start(ch, b):
      pltpu.async_copy(
          src.at[batch, pl.ds(ch * (CHUNK * 3), CHUNK * 3)],
          bufs[b], sems[b])

    def wait(b):
      pltpu.make_async_copy(
          src.at[batch, pl.ds(0, CHUNK * 3)], bufs[b], sems[b]).wait()

    def process(b):
      buf = bufs[b]

      @pl.loop(0, VPC, unroll=8)
      def _vec(v):
        ix = i3 + v * 48
        axes = []
        for d in (0, 1, 2):
          p = plsc.load_gather(buf, [ix + d])
          s = jnp.minimum(jnp.maximum(p * 32.0, _LO), _UP)
          ti = s.astype(jnp.int32)
          tf = ti.astype(jnp.float32)
          blt = s < tf
          frac = (s - tf) + blt.astype(jnp.float32)
          lo = (ti - blt.astype(jnp.int32)) + 32
          w1 = jnp.where(lo < 63, frac, jnp.float32(0.0))
          g1 = jnp.minimum(lo + 1, 63)
          axes.append((lo, g1, jnp.float32(1.0) - frac, w1))
        (lx, gx1, wx0, wx1), (ly, gy1, wy0, wy1), (lz, gz1, wz0, wz1) = axes

        xa = (lx - qlo) * 4096
        xb = (gx1 - qlo) * 4096
        m0 = (lx >= qlo) & (lx < qlo + QPLANES)
        m1 = (gx1 >= qlo) & (gx1 < qlo + QPLANES)
        ya = ly * 64
        yb = gy1 * 64
        yz = ((ya + lz, wy0 * wz0), (ya + gz1, wy0 * wz1),
              (yb + lz, wy1 * wz0), (yb + gz1, wy1 * wz1))
        for xp, wx, m in ((xa, wx0, m0), (xb, wx1, m1)):
          for yzp, wyz in yz:
            plsc.addupdate_scatter(grid_v, [xp + yzp], wx * wyz, mask=m)

    start(0, 0)

    @pl.loop(0, NCH // 2 - 1)
    def _pair(p):
      ch = p * 2
      start(ch + 1, 1)
      wait(0)
      process(0)
      start(ch + 2, 0)
      wait(1)
      process(1)

    start(NCH - 1, 1)
    wait(0)
    process(0)
    wait(1)
    process(1)

    pltpu.sync_copy(grid_v, dst.at[batch, pl.ds(q * QSZ, QSZ)])


@jax.jit
def _gridding(pred, gt):
  mesh = plsc.VectorSubcoreMesh(
      core_axis_name="c", subcore_axis_name="s", num_cores=2, num_subcores=16)
  return pl.kernel(
      _grid_body,
      out_type=(jax.ShapeDtypeStruct((NB, NV), jnp.float32),
                jax.ShapeDtypeStruct((NB, NV), jnp.float32)),
      mesh=mesh,
      compiler_params=pltpu.CompilerParams(needs_layout_passes=False),
      scratch_types=[
          pltpu.VMEM((QSZ,), jnp.float32),
          pltpu.VMEM((CHUNK * 3,), jnp.float32),
          pltpu.VMEM((CHUNK * 3,), jnp.float32),
          pltpu.SemaphoreType.DMA,
          pltpu.SemaphoreType.DMA,
      ],
  )(pred, gt)


def kernel(pred_cloud, gt_cloud):
  pred = pred_cloud.reshape(NB, NPTS * 3)
  gt = gt_cloud.reshape(NB, NPTS * 3)
  return _gridding(pred, gt)
